# Initial kernel scaffold; baseline (speedup 1.0000x reference)
import jax
import jax.numpy as jnp
from jax.experimental import pallas as pl


def kernel(h_node, h_edge, edge_index, relative_vec, distance, node_extra, Wl1, bl1, Wl2, bl2, Wr1, br1, Wr2, br2, Wb, Wn, Wi1, bi1, Wi2, bi2, Wg1, bg1, Wg2, bg2, Wp1, bp1, Wp2, bp2):
    raise NotImplementedError("write your pallas kernel here")


# speedup vs baseline: 2.1392x; 2.1392x over previous
import functools

import jax
import jax.numpy as jnp
from jax import lax
from jax.experimental import pallas as pl
from jax.experimental.pallas import tpu as pltpu
from jax.experimental.pallas import tpu_sc as plsc

_NC, _NS, _L = 2, 16, 16
_NW = _NC * _NS

_G = 80
_TW = 160


def _node_tables_body(h_ref, ex_ref, Wl1, bl1, Wl2, bl2, Wr1, br1, Wr2, br2,
                      Wn_l, Wn_r, Wi1, bi1, Wg1_l, Wg1_r, Wg1_x, bg1,
                      tl_ref, tr_ref):
    h = h_ref[...]
    ex = ex_ref[...]
    Lf = jnp.maximum(h @ Wl1[...] + bl1[...], 0.0) @ Wl2[...] + bl2[...]
    Rf = jnp.maximum(h @ Wr1[...] + br1[...], 0.0) @ Wr2[...] + br2[...]
    WA = Wn_l[...] @ Wi1[...]
    WB = Wn_r[...] @ Wi1[...]
    A = Lf @ WA + bi1[...]
    Bm = Rf @ WB
    Gl = Lf @ Wg1_l[...] + ex @ Wg1_x[...] + bg1[...]
    Gr = Rf @ Wg1_r[...]
    tl_ref[...] = jnp.concatenate([A, Gl], axis=1)
    tr_ref[...] = jnp.concatenate([Bm, Gr], axis=1)


def _full_spec(shape):
    return pl.BlockSpec(shape, lambda i: tuple(0 for _ in shape))


def _node_tables(h_node, node_extra, Wl1, bl1, Wl2, bl2, Wr1, br1, Wr2, br2,
                 Wn_l, Wn_r, Wi1, bi1, Wg1_l, Wg1_r, Wg1_x, bg1):
    n = h_node.shape[0]
    blk = 1000
    grid = (n // blk,)
    ws = [Wl1, bl1, Wl2, bl2, Wr1, br1, Wr2, br2,
          Wn_l, Wn_r, Wi1, bi1, Wg1_l, Wg1_r, Wg1_x, bg1]
    in_specs = [pl.BlockSpec((blk, h_node.shape[1]), lambda i: (i, 0)),
                pl.BlockSpec((blk, node_extra.shape[1]), lambda i: (i, 0))]
    in_specs += [_full_spec(w.shape) for w in ws]
    out_specs = [pl.BlockSpec((blk, _TW), lambda i: (i, 0)),
                 pl.BlockSpec((blk, _TW), lambda i: (i, 0))]
    out_shape = [jax.ShapeDtypeStruct((n, _TW), jnp.float32)] * 2
    return pl.pallas_call(
        _node_tables_body, grid=grid, in_specs=in_specs,
        out_specs=out_specs, out_shape=out_shape,
    )(h_node, node_extra, *ws)


def _edge_uv_body(he_ref, Wb, Wi1, Wg1_e, uv_ref):
    Wu = Wb[...] @ Wi1[...]
    he = he_ref[...]
    uv_ref[...] = jnp.concatenate([he @ Wu, he @ Wg1_e[...]], axis=1)


def _edge_uv(h_edge, Wb, Wi1, Wg1_e):
    e, ed = h_edge.shape
    blk = 4000
    grid = (e // blk,)
    in_specs = [pl.BlockSpec((blk, ed), lambda i: (i, 0)),
                _full_spec(Wb.shape), _full_spec(Wi1.shape),
                _full_spec(Wg1_e.shape)]
    return pl.pallas_call(
        _edge_uv_body, grid=grid, in_specs=in_specs,
        out_specs=pl.BlockSpec((blk, _TW), lambda i: (i, 0)),
        out_shape=jax.ShapeDtypeStruct((e, _TW), jnp.float32),
    )(h_edge, Wb, Wi1, Wg1_e)


def _dfac_body(d_ref, dfac_ref):
    d = d_ref[...]
    dfac_ref[...] = 5.0 / ((d + 1e-6) * (d + 5.0))


def _dfac(distance):
    e = distance.shape[0]
    return pl.pallas_call(
        _dfac_body,
        out_shape=jax.ShapeDtypeStruct((e,), jnp.float32),
    )(distance)


def _sc_edge_body(num_edges, npad,
                  tl_hbm, tr_hbm, uv_hbm, left_hbm, right_hbm,
                  rx_hbm, ry_hbm, rz_hbm,
                  dfac_hbm, wvec_hbm, zeros_hbm, out_hbm,
                  tlb, trb, uvb, rxb, ryb, rzb, dfb, lidx, ridx, wvecb,
                  mbuf, acc, sem):
    cid = lax.axis_index("c")
    sid = lax.axis_index("s")
    wid = sid * _NC + cid
    epw = num_edges // _NW
    base = wid * epw

    pltpu.sync_copy(wvec_hbm, wvecb)
    wi2 = [wvecb[pl.ds(16 * k, 16)] for k in range(8)]
    wg2 = [wvecb[pl.ds(128 + 16 * k, 16)] for k in range(2)]
    bvec = wvecb[pl.ds(160, 16)]
    bi2 = bvec[0]
    bg2 = bvec[1]

    pltpu.sync_copy(zeros_hbm, acc)

    lanes = lax.iota(jnp.int32, 16)
    lanes32 = lanes * 32

    @pl.loop(0, epw // _G)
    def _batch(j):
        eb = base + j * _G
        pltpu.sync_copy(left_hbm.at[pl.ds(eb, _G)], lidx)
        pltpu.sync_copy(right_hbm.at[pl.ds(eb, _G)], ridx)
        cps = [pltpu.async_copy(tl_hbm.at[lidx], tlb, sem),
               pltpu.async_copy(tr_hbm.at[ridx], trb, sem),
               pltpu.async_copy(uv_hbm.at[pl.ds(eb, _G)], uvb, sem),
               pltpu.async_copy(rx_hbm.at[pl.ds(eb, _G)], rxb, sem),
               pltpu.async_copy(ry_hbm.at[pl.ds(eb, _G)], ryb, sem),
               pltpu.async_copy(rz_hbm.at[pl.ds(eb, _G)], rzb, sem),
               pltpu.async_copy(dfac_hbm.at[pl.ds(eb, _G)], dfb, sem)]
        for cp in cps:
            cp.wait()

        @pl.loop(0, _G // 16)
        def _group(g):
            for e in range(16):
                row = g * 16 + e
                xacc = None
                for k in range(8):
                    x = (tlb[row, pl.ds(16 * k, 16)]
                         + trb[row, pl.ds(16 * k, 16)]
                         + uvb[row, pl.ds(16 * k, 16)])
                    t = jnp.maximum(x, 0.0) * wi2[k]
                    xacc = t if xacc is None else xacc + t
                mbuf[pl.ds(32 * e, 16)] = xacc
                gacc = None
                for k in range(2):
                    off = 128 + 16 * k
                    gx = (tlb[row, pl.ds(off, 16)]
                          + trb[row, pl.ds(off, 16)]
                          + uvb[row, pl.ds(off, 16)])
                    t = jnp.maximum(gx, 0.0) * wg2[k]
                    gacc = t if gacc is None else gacc + t
                mbuf[pl.ds(32 * e + 16, 16)] = gacc
            iv = None
            gv = None
            for j in range(16):
                ci = plsc.load_gather(mbuf, [lanes32 + j])
                cg = plsc.load_gather(mbuf, [lanes32 + (16 + j)])
                iv = ci if iv is None else iv + ci
                gv = cg if gv is None else gv + cg
            den = 1.0 + jnp.exp(-(gv + bg2))
            r = 1.0 / den
            r = r * (2.0 - den * r)
            coef = (iv + bi2) * r * dfb[pl.ds(g * 16, 16)]
            li4 = lidx[pl.ds(g * 16, 16)] * 4
            plsc.addupdate_scatter(acc, [li4], coef * rxb[pl.ds(g * 16, 16)])
            plsc.addupdate_scatter(acc, [li4 + 1],
                                   coef * ryb[pl.ds(g * 16, 16)])
            plsc.addupdate_scatter(acc, [li4 + 2],
                                   coef * rzb[pl.ds(g * 16, 16)])

    pltpu.sync_copy(acc, out_hbm.at[wid])


def _sc_edge(tl, tr, uv, left, right, rx, ry, rz, dfac, wvec, zeros, npad):
    num_edges = left.shape[0]
    mesh = plsc.VectorSubcoreMesh(core_axis_name="c", subcore_axis_name="s",
                                  num_cores=_NC, num_subcores=_NS)
    body = functools.partial(_sc_edge_body, num_edges, npad)
    k = pl.kernel(
        body,
        out_type=jax.ShapeDtypeStruct((_NW, npad * 4), jnp.float32),
        mesh=mesh,
        compiler_params=pltpu.CompilerParams(needs_layout_passes=False,
                                             use_tc_tiling_on_sc=False),
        scratch_types=[
            pltpu.VMEM((_G, _TW), jnp.float32),
            pltpu.VMEM((_G, _TW), jnp.float32),
            pltpu.VMEM((_G, _TW), jnp.float32),
            pltpu.VMEM((_G,), jnp.float32),
            pltpu.VMEM((_G,), jnp.float32),
            pltpu.VMEM((_G,), jnp.float32),
            pltpu.VMEM((_G,), jnp.float32),
            pltpu.VMEM((_G,), jnp.int32),
            pltpu.VMEM((_G,), jnp.int32),
            pltpu.VMEM((176,), jnp.float32),
            pltpu.VMEM((512,), jnp.float32),
            pltpu.VMEM((npad * 4,), jnp.float32),
            pltpu.SemaphoreType.DMA,
        ],
    )
    return k(tl, tr, uv, left, right, rx, ry, rz, dfac, wvec, zeros)


def _finish_body(dp_ref, h_ref, ex_ref, Wp_h, Wp_x, wp_d, bp1, Wp2, bp2,
                 out_ref):
    dp = jnp.sum(dp_ref[...], axis=0)
    dn = jnp.sqrt(jnp.sum(dp * dp, axis=1, keepdims=True))
    pre = (h_ref[...] @ Wp_h[...] + ex_ref[...] @ Wp_x[...]
           + dn * wp_d[...] + bp1[...])
    s = jnp.maximum(pre, 0.0) @ Wp2[...] + bp2[...]
    scale = 1.0 / (1.0 + jnp.exp(-s))
    out_ref[...] = dp[:, :3] * scale


def _finish(dpart, h_node, node_extra, Wp_h, Wp_x, wp_d, bp1, Wp2, bp2):
    n = h_node.shape[0]
    blk = 1000
    grid = (n // blk,)
    ws = [Wp_h, Wp_x, wp_d, bp1, Wp2, bp2]
    in_specs = [pl.BlockSpec((_NW, blk, 4), lambda i: (0, i, 0)),
                pl.BlockSpec((blk, h_node.shape[1]), lambda i: (i, 0)),
                pl.BlockSpec((blk, node_extra.shape[1]), lambda i: (i, 0))]
    in_specs += [_full_spec(w.shape) for w in ws]
    return pl.pallas_call(
        _finish_body, grid=grid, in_specs=in_specs,
        out_specs=pl.BlockSpec((blk, 3), lambda i: (i, 0)),
        out_shape=jax.ShapeDtypeStruct((n, 3), jnp.float32),
    )(dpart, h_node, node_extra, *ws)


def kernel(h_node, h_edge, edge_index, relative_vec, distance, node_extra,
           Wl1, bl1, Wl2, bl2, Wr1, br1, Wr2, br2,
           Wb, Wn, Wi1, bi1, Wi2, bi2, Wg1, bg1, Wg2, bg2,
           Wp1, bp1, Wp2, bp2):
    n, nd = h_node.shape
    e, ed = h_edge.shape
    npad = ((n + 8 * _NS - 1) // (8 * _NS)) * (8 * _NS)

    left = edge_index[0].astype(jnp.int32)
    right = edge_index[1].astype(jnp.int32)

    Wn_l, Wn_r = Wn[:nd], Wn[nd:]
    Wg1_e = Wg1[:ed]
    Wg1_l = Wg1[ed:ed + nd]
    Wg1_r = Wg1[ed + nd:ed + 2 * nd]
    Wg1_x = Wg1[ed + 2 * nd:]
    wvec = jnp.concatenate([
        Wi2[:, 0], Wg2[:, 0], bi2, bg2,
        jnp.zeros((176 - nd - 32 - 2,), jnp.float32)])
    zeros = jnp.zeros((npad * 4,), jnp.float32)
    Wp_h = Wp1[:nd]
    Wp_x = Wp1[nd:nd + 2]
    wp_d = Wp1[nd + 2:nd + 3]

    tl, tr = _node_tables(h_node, node_extra, Wl1, bl1, Wl2, bl2,
                          Wr1, br1, Wr2, br2, Wn_l, Wn_r, Wi1, bi1,
                          Wg1_l, Wg1_r, Wg1_x, bg1)
    uv = _edge_uv(h_edge, Wb, Wi1, Wg1_e)
    dfac = _dfac(distance)
    rx, ry, rz = relative_vec[:, 0], relative_vec[:, 1], relative_vec[:, 2]
    parts = _sc_edge(tl, tr, uv, left, right, rx, ry, rz, dfac,
                     wvec, zeros, npad)
    dpart = parts.reshape(_NW, npad, 4)
    return _finish(dpart, h_node, node_extra, Wp_h, Wp_x, wp_d, bp1, Wp2, bp2)


# speedup vs baseline: 2.2839x; 1.0676x over previous
import functools

import jax
import jax.numpy as jnp
from jax import lax
from jax.experimental import pallas as pl
from jax.experimental.pallas import tpu as pltpu
from jax.experimental.pallas import tpu_sc as plsc

def _dot(a, b):
    return jnp.dot(a, b, precision=jax.lax.Precision.HIGHEST)


_NC, _NS, _L = 2, 16, 16
_NW = _NC * _NS

_G = 80
_TW = 160


def _node_tables_body(h_ref, ex_ref, Wl1, bl1, Wl2, bl2, Wr1, br1, Wr2, br2,
                      Wn_l, Wn_r, Wi1, bi1, Wg1_l, Wg1_r, Wg1_x, bg1,
                      tl_ref, tr_ref):
    h = h_ref[...]
    ex = ex_ref[...]
    Lf = _dot(jnp.maximum(_dot(h, Wl1[...]) + bl1[...], 0.0),
              Wl2[...]) + bl2[...]
    Rf = _dot(jnp.maximum(_dot(h, Wr1[...]) + br1[...], 0.0),
              Wr2[...]) + br2[...]
    WA = _dot(Wn_l[...], Wi1[...])
    WB = _dot(Wn_r[...], Wi1[...])
    A = _dot(Lf, WA) + bi1[...]
    Bm = _dot(Rf, WB)
    Gl = _dot(Lf, Wg1_l[...]) + _dot(ex, Wg1_x[...]) + bg1[...]
    Gr = _dot(Rf, Wg1_r[...])
    tl_ref[...] = jnp.concatenate([A, Gl], axis=1)
    tr_ref[...] = jnp.concatenate([Bm, Gr], axis=1)


def _full_spec(shape):
    return pl.BlockSpec(shape, lambda i: tuple(0 for _ in shape))


def _node_tables(h_node, node_extra, Wl1, bl1, Wl2, bl2, Wr1, br1, Wr2, br2,
                 Wn_l, Wn_r, Wi1, bi1, Wg1_l, Wg1_r, Wg1_x, bg1):
    n = h_node.shape[0]
    blk = 1000
    grid = (n // blk,)
    ws = [Wl1, bl1, Wl2, bl2, Wr1, br1, Wr2, br2,
          Wn_l, Wn_r, Wi1, bi1, Wg1_l, Wg1_r, Wg1_x, bg1]
    in_specs = [pl.BlockSpec((blk, h_node.shape[1]), lambda i: (i, 0)),
                pl.BlockSpec((blk, node_extra.shape[1]), lambda i: (i, 0))]
    in_specs += [_full_spec(w.shape) for w in ws]
    out_specs = [pl.BlockSpec((blk, _TW), lambda i: (i, 0)),
                 pl.BlockSpec((blk, _TW), lambda i: (i, 0))]
    out_shape = [jax.ShapeDtypeStruct((n, _TW), jnp.float32)] * 2
    return pl.pallas_call(
        _node_tables_body, grid=grid, in_specs=in_specs,
        out_specs=out_specs, out_shape=out_shape,
    )(h_node, node_extra, *ws)


def _edge_uv_body(blk, he_ref, Wb, Wi1, Wg1_e, uv_ref, uvg_ref):
    Wu = _dot(Wb[...], Wi1[...])
    he = he_ref[...]
    uv_ref[...] = _dot(he, Wu)
    uvg_ref[...] = jnp.concatenate(
        [_dot(he, Wg1_e[...]), jnp.zeros((blk, 96), jnp.float32)], axis=1)


def _edge_uv(h_edge, Wb, Wi1, Wg1_e):
    e, ed = h_edge.shape
    blk = 4000
    grid = (e // blk,)
    in_specs = [pl.BlockSpec((blk, ed), lambda i: (i, 0)),
                _full_spec(Wb.shape), _full_spec(Wi1.shape),
                _full_spec(Wg1_e.shape)]
    return pl.pallas_call(
        functools.partial(_edge_uv_body, blk), grid=grid, in_specs=in_specs,
        out_specs=[pl.BlockSpec((blk, 128), lambda i: (i, 0)),
                   pl.BlockSpec((blk, 128), lambda i: (i, 0))],
        out_shape=[jax.ShapeDtypeStruct((e, 128), jnp.float32),
                   jax.ShapeDtypeStruct((e, 128), jnp.float32)],
    )(h_edge, Wb, Wi1, Wg1_e)


def _dfac_body(d_ref, dfac_ref):
    d = d_ref[...]
    dfac_ref[...] = 5.0 / ((d + 1e-6) * (d + 5.0))


def _dfac(distance):
    e = distance.shape[0]
    return pl.pallas_call(
        _dfac_body,
        out_shape=jax.ShapeDtypeStruct((e,), jnp.float32),
    )(distance)


def _sc_edge_body(num_edges, npad,
                  tl_hbm, tr_hbm, uv_hbm, uvg_hbm, left_hbm, right_hbm,
                  rx_hbm, ry_hbm, rz_hbm,
                  dfac_hbm, wvec_hbm, zeros_hbm, out_hbm,
                  tlb, trb, uvb, uvgb, rxb, ryb, rzb, dfb, lidx, ridx, wvecb,
                  mbuf, acc, sem):
    cid = lax.axis_index("c")
    sid = lax.axis_index("s")
    wid = sid * _NC + cid
    epw = num_edges // _NW
    base = wid * epw

    pltpu.sync_copy(wvec_hbm, wvecb)
    wi2 = [wvecb[pl.ds(16 * k, 16)] for k in range(8)]
    wg2 = [wvecb[pl.ds(128 + 16 * k, 16)] for k in range(2)]
    bvec = wvecb[pl.ds(160, 16)]
    bi2 = bvec[0]
    bg2 = bvec[1]

    pltpu.sync_copy(zeros_hbm, acc)

    lanes = lax.iota(jnp.int32, 16)
    lanes32 = lanes * 32

    @pl.loop(0, epw // _G)
    def _batch(j):
        eb = base + j * _G
        pltpu.sync_copy(left_hbm.at[pl.ds(eb, _G)], lidx)
        pltpu.sync_copy(right_hbm.at[pl.ds(eb, _G)], ridx)
        cps = [pltpu.async_copy(tl_hbm.at[lidx], tlb, sem),
               pltpu.async_copy(tr_hbm.at[ridx], trb, sem),
               pltpu.async_copy(uv_hbm.at[pl.ds(eb, _G)], uvb, sem),
               pltpu.async_copy(uvg_hbm.at[pl.ds(eb, _G)], uvgb, sem),
               pltpu.async_copy(rx_hbm.at[pl.ds(eb, _G)], rxb, sem),
               pltpu.async_copy(ry_hbm.at[pl.ds(eb, _G)], ryb, sem),
               pltpu.async_copy(rz_hbm.at[pl.ds(eb, _G)], rzb, sem),
               pltpu.async_copy(dfac_hbm.at[pl.ds(eb, _G)], dfb, sem)]
        for cp in cps:
            cp.wait()

        @pl.loop(0, _G // 16)
        def _group(g):
            for e in range(16):
                row = g * 16 + e
                xacc = None
                for k in range(8):
                    x = (tlb[row, pl.ds(16 * k, 16)]
                         + trb[row, pl.ds(16 * k, 16)]
                         + uvb[row, pl.ds(16 * k, 16)])
                    t = jnp.maximum(x, 0.0) * wi2[k]
                    xacc = t if xacc is None else xacc + t
                mbuf[pl.ds(32 * e, 16)] = xacc
                gacc = None
                for k in range(2):
                    off = 128 + 16 * k
                    gx = (tlb[row, pl.ds(off, 16)]
                          + trb[row, pl.ds(off, 16)]
                          + uvgb[row, pl.ds(16 * k, 16)])
                    t = jnp.maximum(gx, 0.0) * wg2[k]
                    gacc = t if gacc is None else gacc + t
                mbuf[pl.ds(32 * e + 16, 16)] = gacc
            iv = None
            gv = None
            for j in range(16):
                ci = plsc.load_gather(mbuf, [lanes32 + j])
                cg = plsc.load_gather(mbuf, [lanes32 + (16 + j)])
                iv = ci if iv is None else iv + ci
                gv = cg if gv is None else gv + cg
            den = 1.0 + jnp.exp(-(gv + bg2))
            r = 1.0 / den
            r = r * (2.0 - den * r)
            coef = (iv + bi2) * r * dfb[pl.ds(g * 16, 16)]
            li4 = lidx[pl.ds(g * 16, 16)] * 4
            plsc.addupdate_scatter(acc, [li4], coef * rxb[pl.ds(g * 16, 16)])
            plsc.addupdate_scatter(acc, [li4 + 1],
                                   coef * ryb[pl.ds(g * 16, 16)])
            plsc.addupdate_scatter(acc, [li4 + 2],
                                   coef * rzb[pl.ds(g * 16, 16)])

    pltpu.sync_copy(acc, out_hbm.at[wid])


def _sc_edge(tl, tr, uv, uvg, left, right, rx, ry, rz, dfac, wvec, zeros,
             npad):
    num_edges = left.shape[0]
    mesh = plsc.VectorSubcoreMesh(core_axis_name="c", subcore_axis_name="s",
                                  num_cores=_NC, num_subcores=_NS)
    body = functools.partial(_sc_edge_body, num_edges, npad)
    k = pl.kernel(
        body,
        out_type=jax.ShapeDtypeStruct((_NW, npad * 4), jnp.float32),
        mesh=mesh,
        compiler_params=pltpu.CompilerParams(needs_layout_passes=False,
                                             use_tc_tiling_on_sc=False),
        scratch_types=[
            pltpu.VMEM((_G, _TW), jnp.float32),
            pltpu.VMEM((_G, _TW), jnp.float32),
            pltpu.VMEM((_G, 128), jnp.float32),
            pltpu.VMEM((_G, 128), jnp.float32),
            pltpu.VMEM((_G,), jnp.float32),
            pltpu.VMEM((_G,), jnp.float32),
            pltpu.VMEM((_G,), jnp.float32),
            pltpu.VMEM((_G,), jnp.float32),
            pltpu.VMEM((_G,), jnp.int32),
            pltpu.VMEM((_G,), jnp.int32),
            pltpu.VMEM((176,), jnp.float32),
            pltpu.VMEM((512,), jnp.float32),
            pltpu.VMEM((npad * 4,), jnp.float32),
            pltpu.SemaphoreType.DMA,
        ],
    )
    return k(tl, tr, uv, uvg, left, right, rx, ry, rz, dfac, wvec, zeros)


def _finish_body(dp_ref, h_ref, ex_ref, Wp_h, Wp_x, wp_d, bp1, Wp2, bp2,
                 out_ref):
    dp = jnp.sum(dp_ref[...], axis=0)
    dn = jnp.sqrt(jnp.sum(dp * dp, axis=1, keepdims=True))
    pre = (_dot(h_ref[...], Wp_h[...]) + _dot(ex_ref[...], Wp_x[...])
           + dn * wp_d[...] + bp1[...])
    s = _dot(jnp.maximum(pre, 0.0), Wp2[...]) + bp2[...]
    scale = 1.0 / (1.0 + jnp.exp(-s))
    out_ref[...] = dp[:, :3] * scale


def _finish(dpart, h_node, node_extra, Wp_h, Wp_x, wp_d, bp1, Wp2, bp2):
    n = h_node.shape[0]
    blk = 1000
    grid = (n // blk,)
    ws = [Wp_h, Wp_x, wp_d, bp1, Wp2, bp2]
    in_specs = [pl.BlockSpec((_NW, blk, 4), lambda i: (0, i, 0)),
                pl.BlockSpec((blk, h_node.shape[1]), lambda i: (i, 0)),
                pl.BlockSpec((blk, node_extra.shape[1]), lambda i: (i, 0))]
    in_specs += [_full_spec(w.shape) for w in ws]
    return pl.pallas_call(
        _finish_body, grid=grid, in_specs=in_specs,
        out_specs=pl.BlockSpec((blk, 3), lambda i: (i, 0)),
        out_shape=jax.ShapeDtypeStruct((n, 3), jnp.float32),
    )(dpart, h_node, node_extra, *ws)


def kernel(h_node, h_edge, edge_index, relative_vec, distance, node_extra,
           Wl1, bl1, Wl2, bl2, Wr1, br1, Wr2, br2,
           Wb, Wn, Wi1, bi1, Wi2, bi2, Wg1, bg1, Wg2, bg2,
           Wp1, bp1, Wp2, bp2):
    n, nd = h_node.shape
    e, ed = h_edge.shape
    npad = ((n + 8 * _NS - 1) // (8 * _NS)) * (8 * _NS)

    left = edge_index[0].astype(jnp.int32)
    right = edge_index[1].astype(jnp.int32)

    Wn_l, Wn_r = Wn[:nd], Wn[nd:]
    Wg1_e = Wg1[:ed]
    Wg1_l = Wg1[ed:ed + nd]
    Wg1_r = Wg1[ed + nd:ed + 2 * nd]
    Wg1_x = Wg1[ed + 2 * nd:]
    wvec = jnp.concatenate([
        Wi2[:, 0], Wg2[:, 0], bi2, bg2,
        jnp.zeros((176 - nd - 32 - 2,), jnp.float32)])
    zeros = jnp.zeros((npad * 4,), jnp.float32)
    Wp_h = Wp1[:nd]
    Wp_x = Wp1[nd:nd + 2]
    wp_d = Wp1[nd + 2:nd + 3]

    tl, tr = _node_tables(h_node, node_extra, Wl1, bl1, Wl2, bl2,
                          Wr1, br1, Wr2, br2, Wn_l, Wn_r, Wi1, bi1,
                          Wg1_l, Wg1_r, Wg1_x, bg1)
    uv, uvg = _edge_uv(h_edge, Wb, Wi1, Wg1_e)
    dfac = _dfac(distance)
    rx, ry, rz = relative_vec[:, 0], relative_vec[:, 1], relative_vec[:, 2]
    parts = _sc_edge(tl, tr, uv, uvg, left, right, rx, ry, rz, dfac,
                     wvec, zeros, npad)
    dpart = parts.reshape(_NW, npad, 4)
    return _finish(dpart, h_node, node_extra, Wp_h, Wp_x, wp_d, bp1, Wp2, bp2)


# speedup vs baseline: 2.5764x; 1.1281x over previous
import functools

import jax
import jax.numpy as jnp
from jax import lax
from jax.experimental import pallas as pl
from jax.experimental.pallas import tpu as pltpu
from jax.experimental.pallas import tpu_sc as plsc

def _dot(a, b):
    return jnp.dot(a, b, precision=jax.lax.Precision.HIGHEST)


_NC, _NS, _L = 2, 16, 16
_NW = _NC * _NS

_G = 80
_TW = 160


def _node_tables_body(h_ref, ex_ref, Wl1, bl1, Wl2, bl2, Wr1, br1, Wr2, br2,
                      Wn_l, Wn_r, Wi1, bi1, Wg1_l, Wg1_r, Wg1_x, bg1,
                      tl_ref, tr_ref):
    h = h_ref[...]
    ex = ex_ref[...]
    Lf = _dot(jnp.maximum(_dot(h, Wl1[...]) + bl1[...], 0.0),
              Wl2[...]) + bl2[...]
    Rf = _dot(jnp.maximum(_dot(h, Wr1[...]) + br1[...], 0.0),
              Wr2[...]) + br2[...]
    WA = _dot(Wn_l[...], Wi1[...])
    WB = _dot(Wn_r[...], Wi1[...])
    A = _dot(Lf, WA) + bi1[...]
    Bm = _dot(Rf, WB)
    Gl = _dot(Lf, Wg1_l[...]) + _dot(ex, Wg1_x[...]) + bg1[...]
    Gr = _dot(Rf, Wg1_r[...])
    tl_ref[...] = jnp.concatenate([A, Gl], axis=1)
    tr_ref[...] = jnp.concatenate([Bm, Gr], axis=1)


def _full_spec(shape):
    return pl.BlockSpec(shape, lambda i: tuple(0 for _ in shape))


def _node_tables(h_node, node_extra, Wl1, bl1, Wl2, bl2, Wr1, br1, Wr2, br2,
                 Wn_l, Wn_r, Wi1, bi1, Wg1_l, Wg1_r, Wg1_x, bg1):
    n = h_node.shape[0]
    blk = 1000
    grid = (n // blk,)
    ws = [Wl1, bl1, Wl2, bl2, Wr1, br1, Wr2, br2,
          Wn_l, Wn_r, Wi1, bi1, Wg1_l, Wg1_r, Wg1_x, bg1]
    in_specs = [pl.BlockSpec((blk, h_node.shape[1]), lambda i: (i, 0)),
                pl.BlockSpec((blk, node_extra.shape[1]), lambda i: (i, 0))]
    in_specs += [_full_spec(w.shape) for w in ws]
    out_specs = [pl.BlockSpec((blk, _TW), lambda i: (i, 0)),
                 pl.BlockSpec((blk, _TW), lambda i: (i, 0))]
    out_shape = [jax.ShapeDtypeStruct((n, _TW), jnp.float32)] * 2
    return pl.pallas_call(
        _node_tables_body, grid=grid, in_specs=in_specs,
        out_specs=out_specs, out_shape=out_shape,
    )(h_node, node_extra, *ws)


def _edge_uv_body(blk, he_ref, Wb, Wi1, Wg1_e, uv_ref, uvg_ref):
    Wu = _dot(Wb[...], Wi1[...])
    he = he_ref[...]
    uv_ref[...] = _dot(he, Wu)
    uvg_ref[...] = jnp.concatenate(
        [_dot(he, Wg1_e[...]), jnp.zeros((blk, 96), jnp.float32)], axis=1)


def _edge_uv(h_edge, Wb, Wi1, Wg1_e):
    e, ed = h_edge.shape
    blk = 4000
    grid = (e // blk,)
    in_specs = [pl.BlockSpec((blk, ed), lambda i: (i, 0)),
                _full_spec(Wb.shape), _full_spec(Wi1.shape),
                _full_spec(Wg1_e.shape)]
    return pl.pallas_call(
        functools.partial(_edge_uv_body, blk), grid=grid, in_specs=in_specs,
        out_specs=[pl.BlockSpec((blk, 128), lambda i: (i, 0)),
                   pl.BlockSpec((blk, 128), lambda i: (i, 0))],
        out_shape=[jax.ShapeDtypeStruct((e, 128), jnp.float32),
                   jax.ShapeDtypeStruct((e, 128), jnp.float32)],
    )(h_edge, Wb, Wi1, Wg1_e)


def _dfac_body(d_ref, dfac_ref):
    d = d_ref[...]
    dfac_ref[...] = 5.0 / ((d + 1e-6) * (d + 5.0))


def _dfac(distance):
    e = distance.shape[0]
    return pl.pallas_call(
        _dfac_body,
        out_shape=jax.ShapeDtypeStruct((e,), jnp.float32),
    )(distance)


def _sc_edge_body(num_edges, npad,
                  tl_hbm, tr_hbm, uv_hbm, uvg_hbm, left_hbm, right_hbm,
                  rx_hbm, ry_hbm, rz_hbm,
                  dfac_hbm, wvec_hbm, zeros_hbm, out_hbm,
                  tlbA, trbA, uvbA, uvgbA, lidxA, ridxA,
                  tlbB, trbB, uvbB, uvgbB, lidxB, ridxB,
                  rxb, ryb, rzb, dfb, wvecb, mbuf, acc, semA, semB):
    cid = lax.axis_index("c")
    sid = lax.axis_index("s")
    wid = sid * _NC + cid
    epw = num_edges // _NW
    base = wid * epw
    nb = epw // _G

    pltpu.sync_copy(wvec_hbm, wvecb)
    wi2 = [wvecb[pl.ds(16 * k, 16)] for k in range(8)]
    wg2 = [wvecb[pl.ds(128 + 16 * k, 16)] for k in range(2)]
    bvec = wvecb[pl.ds(160, 16)]
    bi2 = bvec[0]
    bg2 = bvec[1]

    pltpu.sync_copy(zeros_hbm, acc)

    lanes = lax.iota(jnp.int32, 16)
    lanes32 = lanes * 32

    bufsA = (tlbA, trbA, uvbA, uvgbA, lidxA, ridxA)
    bufsB = (tlbB, trbB, uvbB, uvgbB, lidxB, ridxB)

    def start(j, bufs, sem):
        tlb, trb, uvb, uvgb, lidx, ridx = bufs
        eb = base + j * _G
        pltpu.sync_copy(left_hbm.at[pl.ds(eb, _G)], lidx)
        pltpu.sync_copy(right_hbm.at[pl.ds(eb, _G)], ridx)
        pltpu.async_copy(tl_hbm.at[lidx], tlb, sem)
        pltpu.async_copy(tr_hbm.at[ridx], trb, sem)
        pltpu.async_copy(uv_hbm.at[pl.ds(eb, _G)], uvb, sem)
        pltpu.async_copy(uvg_hbm.at[pl.ds(eb, _G)], uvgb, sem)

    def drain(bufs, sem):
        tlb, trb, uvb, uvgb, _, _ = bufs
        pltpu.make_async_copy(tl_hbm.at[pl.ds(0, _G)], tlb, sem).wait()
        pltpu.make_async_copy(tr_hbm.at[pl.ds(0, _G)], trb, sem).wait()
        pltpu.make_async_copy(uv_hbm.at[pl.ds(0, _G)], uvb, sem).wait()
        pltpu.make_async_copy(uvg_hbm.at[pl.ds(0, _G)], uvgb, sem).wait()

    def compute(j, bufs):
        tlb, trb, uvb, uvgb, lidx, ridx = bufs
        eb = base + j * _G
        pltpu.sync_copy(rx_hbm.at[pl.ds(eb, _G)], rxb)
        pltpu.sync_copy(ry_hbm.at[pl.ds(eb, _G)], ryb)
        pltpu.sync_copy(rz_hbm.at[pl.ds(eb, _G)], rzb)
        pltpu.sync_copy(dfac_hbm.at[pl.ds(eb, _G)], dfb)

        @pl.loop(0, _G // 16)
        def _group(g):
            for e in range(16):
                row = g * 16 + e
                xacc = None
                for k in range(8):
                    x = (tlb[row, pl.ds(16 * k, 16)]
                         + trb[row, pl.ds(16 * k, 16)]
                         + uvb[row, pl.ds(16 * k, 16)])
                    t = jnp.maximum(x, 0.0) * wi2[k]
                    xacc = t if xacc is None else xacc + t
                mbuf[pl.ds(32 * e, 16)] = xacc
                gacc = None
                for k in range(2):
                    off = 128 + 16 * k
                    gx = (tlb[row, pl.ds(off, 16)]
                          + trb[row, pl.ds(off, 16)]
                          + uvgb[row, pl.ds(16 * k, 16)])
                    t = jnp.maximum(gx, 0.0) * wg2[k]
                    gacc = t if gacc is None else gacc + t
                mbuf[pl.ds(32 * e + 16, 16)] = gacc
            iv = None
            gv = None
            for j2 in range(16):
                ci = plsc.load_gather(mbuf, [lanes32 + j2])
                cg = plsc.load_gather(mbuf, [lanes32 + (16 + j2)])
                iv = ci if iv is None else iv + ci
                gv = cg if gv is None else gv + cg
            den = 1.0 + jnp.exp(-(gv + bg2))
            r = 1.0 / den
            r = r * (2.0 - den * r)
            coef = (iv + bi2) * r * dfb[pl.ds(g * 16, 16)]
            li = lidx[pl.ds(g * 16, 16)]
            plsc.addupdate_scatter(acc, [li], coef * rxb[pl.ds(g * 16, 16)])
            plsc.addupdate_scatter(acc, [li + npad],
                                   coef * ryb[pl.ds(g * 16, 16)])
            plsc.addupdate_scatter(acc, [li + 2 * npad],
                                   coef * rzb[pl.ds(g * 16, 16)])

    start(0, bufsA, semA)

    @pl.loop(0, (nb - 1) // 2)
    def _pair(p):
        jA = 2 * p
        start(jA + 1, bufsB, semB)
        drain(bufsA, semA)
        compute(jA, bufsA)
        start(jA + 2, bufsA, semA)
        drain(bufsB, semB)
        compute(jA + 1, bufsB)

    drain(bufsA, semA)
    compute(nb - 1, bufsA)

    pltpu.sync_copy(acc, out_hbm.at[wid])


def _sc_edge(tl, tr, uv, uvg, left, right, rx, ry, rz, dfac, wvec, zeros,
             npad):
    num_edges = left.shape[0]
    mesh = plsc.VectorSubcoreMesh(core_axis_name="c", subcore_axis_name="s",
                                  num_cores=_NC, num_subcores=_NS)
    body = functools.partial(_sc_edge_body, num_edges, npad)
    stream_bufs = [
        pltpu.VMEM((_G, _TW), jnp.float32),
        pltpu.VMEM((_G, _TW), jnp.float32),
        pltpu.VMEM((_G, 128), jnp.float32),
        pltpu.VMEM((_G, 128), jnp.float32),
        pltpu.VMEM((_G,), jnp.int32),
        pltpu.VMEM((_G,), jnp.int32),
    ]
    k = pl.kernel(
        body,
        out_type=jax.ShapeDtypeStruct((_NW, npad * 3), jnp.float32),
        mesh=mesh,
        compiler_params=pltpu.CompilerParams(needs_layout_passes=False,
                                             use_tc_tiling_on_sc=False),
        scratch_types=stream_bufs + stream_bufs + [
            pltpu.VMEM((_G,), jnp.float32),
            pltpu.VMEM((_G,), jnp.float32),
            pltpu.VMEM((_G,), jnp.float32),
            pltpu.VMEM((_G,), jnp.float32),
            pltpu.VMEM((176,), jnp.float32),
            pltpu.VMEM((512,), jnp.float32),
            pltpu.VMEM((npad * 3,), jnp.float32),
            pltpu.SemaphoreType.DMA,
            pltpu.SemaphoreType.DMA,
        ],
    )
    return k(tl, tr, uv, uvg, left, right, rx, ry, rz, dfac, wvec, zeros)


def _finish_body(dp_ref, h_ref, ex_ref, Wp_h, Wp_x, wp_d, bp1, Wp2, bp2,
                 out_ref):
    s = jnp.sum(dp_ref[...], axis=0)
    px = s[0]
    py = s[1]
    pz = s[2]
    dn = jnp.sqrt(px * px + py * py + pz * pz)[:, None]
    pre = (_dot(h_ref[...], Wp_h[...]) + _dot(ex_ref[...], Wp_x[...])
           + dn * wp_d[...] + bp1[...])
    s2 = _dot(jnp.maximum(pre, 0.0), Wp2[...]) + bp2[...]
    scale = 1.0 / (1.0 + jnp.exp(-s2))
    out_ref[...] = jnp.stack([px, py, pz], axis=1) * scale


def _finish(dpart, h_node, node_extra, Wp_h, Wp_x, wp_d, bp1, Wp2, bp2):
    n = h_node.shape[0]
    blk = 1024
    grid = ((n + blk - 1) // blk,)
    ws = [Wp_h, Wp_x, wp_d, bp1, Wp2, bp2]
    in_specs = [pl.BlockSpec((_NW, 3, blk), lambda i: (0, 0, i)),
                pl.BlockSpec((blk, h_node.shape[1]), lambda i: (i, 0)),
                pl.BlockSpec((blk, node_extra.shape[1]), lambda i: (i, 0))]
    in_specs += [_full_spec(w.shape) for w in ws]
    return pl.pallas_call(
        _finish_body, grid=grid, in_specs=in_specs,
        out_specs=pl.BlockSpec((blk, 3), lambda i: (i, 0)),
        out_shape=jax.ShapeDtypeStruct((n, 3), jnp.float32),
    )(dpart, h_node, node_extra, *ws)


def kernel(h_node, h_edge, edge_index, relative_vec, distance, node_extra,
           Wl1, bl1, Wl2, bl2, Wr1, br1, Wr2, br2,
           Wb, Wn, Wi1, bi1, Wi2, bi2, Wg1, bg1, Wg2, bg2,
           Wp1, bp1, Wp2, bp2):
    n, nd = h_node.shape
    e, ed = h_edge.shape
    npad = ((n + 8 * _NS - 1) // (8 * _NS)) * (8 * _NS)

    left = edge_index[0].astype(jnp.int32)
    right = edge_index[1].astype(jnp.int32)

    Wn_l, Wn_r = Wn[:nd], Wn[nd:]
    Wg1_e = Wg1[:ed]
    Wg1_l = Wg1[ed:ed + nd]
    Wg1_r = Wg1[ed + nd:ed + 2 * nd]
    Wg1_x = Wg1[ed + 2 * nd:]
    wvec = jnp.concatenate([
        Wi2[:, 0], Wg2[:, 0], bi2, bg2,
        jnp.zeros((176 - nd - 32 - 2,), jnp.float32)])
    zeros = jnp.zeros((npad * 3,), jnp.float32)
    Wp_h = Wp1[:nd]
    Wp_x = Wp1[nd:nd + 2]
    wp_d = Wp1[nd + 2:nd + 3]

    tl, tr = _node_tables(h_node, node_extra, Wl1, bl1, Wl2, bl2,
                          Wr1, br1, Wr2, br2, Wn_l, Wn_r, Wi1, bi1,
                          Wg1_l, Wg1_r, Wg1_x, bg1)
    uv, uvg = _edge_uv(h_edge, Wb, Wi1, Wg1_e)
    dfac = _dfac(distance)
    rx, ry, rz = relative_vec[:, 0], relative_vec[:, 1], relative_vec[:, 2]
    parts = _sc_edge(tl, tr, uv, uvg, left, right, rx, ry, rz, dfac,
                     wvec, zeros, npad)
    dpart = parts.reshape(_NW, 3, npad)
    return _finish(dpart, h_node, node_extra, Wp_h, Wp_x, wp_d, bp1, Wp2, bp2)
